# baseline (speedup 1.0000x reference)
import jax
import jax.numpy as jnp
from jax.experimental import pallas as pl

_TAU = 0.5


def _body(p_ref, o_ref):
    o_ref[...] = p_ref[...] >= _TAU


def kernel(priority):
    p = jnp.squeeze(priority, axis=-1)
    rows, cols = p.shape
    block_rows = 16
    return pl.pallas_call(
        _body,
        grid=(rows // block_rows,),
        in_specs=[pl.BlockSpec((block_rows, cols), lambda i: (i, 0))],
        out_specs=pl.BlockSpec((block_rows, cols), lambda i: (i, 0)),
        out_shape=jax.ShapeDtypeStruct((rows, cols), jnp.bool_),
    )(p)


# speedup vs baseline: 1.1520x; 1.1520x over previous
import jax
import jax.numpy as jnp
from jax.experimental import pallas as pl

_TAU = 0.5


def _body(p_ref, o_ref):
    o_ref[...] = (p_ref[...] >= _TAU).astype(jnp.uint8)


def kernel(priority):
    rows, cols, _ = priority.shape
    p = priority.reshape(rows, cols)
    block_rows = 32
    y = pl.pallas_call(
        _body,
        grid=(rows // block_rows,),
        in_specs=[pl.BlockSpec((block_rows, cols), lambda i: (i, 0))],
        out_specs=pl.BlockSpec((block_rows, cols), lambda i: (i, 0)),
        out_shape=jax.ShapeDtypeStruct((rows, cols), jnp.uint8),
    )(p)
    return y.view(jnp.bool_)


# speedup vs baseline: 3.3570x; 2.9140x over previous
import jax
import jax.numpy as jnp
from jax.experimental import pallas as pl

_TAU = 0.5


def _body(p_ref, o_ref):
    m = (p_ref[...] >= _TAU).astype(jnp.uint8)
    o_ref[...] = m.reshape(o_ref.shape)


def kernel(priority):
    rows, cols, _ = priority.shape
    lanes = 128
    sub = cols // lanes
    x = priority.reshape(rows, sub, lanes)
    block_rows = 32
    grid = rows // block_rows
    y = pl.pallas_call(
        _body,
        grid=(grid,),
        in_specs=[pl.BlockSpec((block_rows, sub, lanes), lambda i: (i, 0, 0))],
        out_specs=pl.BlockSpec((block_rows, cols), lambda i: (i, 0)),
        out_shape=jax.ShapeDtypeStruct((rows, cols), jnp.uint8),
    )(x)
    return y != 0
